# baseline (speedup 1.0000x reference)
import functools

import jax
import jax.numpy as jnp
from jax import lax
from jax.experimental import pallas as pl
from jax.experimental.pallas import tpu as pltpu
from jax.experimental.pallas import tpu_sc as plsc

VOCAB = 32000
HIDDEN = 2048
BATCH = 4
SEQ = 4096

NUM_CORES = 2
NUM_SUBCORES = 16
NUM_WORKERS = NUM_CORES * NUM_SUBCORES
ROWS = BATCH * SEQ
ROWS_PER_W = ROWS // NUM_WORKERS
CHUNK = 16
N_CHUNKS = ROWS_PER_W // CHUNK
NBUF = 3
K = 1


def _make_gather():
  mesh = plsc.VectorSubcoreMesh(
      core_axis_name="c", subcore_axis_name="s",
      num_cores=NUM_CORES, num_subcores=NUM_SUBCORES)

  S_PER_W = ROWS_PER_W // BATCH

  @functools.partial(
      pl.kernel,
      out_type=jax.ShapeDtypeStruct((SEQ, BATCH, HIDDEN), jnp.float32),
      mesh=mesh,
      scratch_types=[
          pltpu.VMEM((ROWS_PER_W,), jnp.int32),
          [pltpu.VMEM((CHUNK, HIDDEN), jnp.float32) for _ in range(NBUF)],
          [pltpu.SemaphoreType.DMA for _ in range(NBUF)],
          [pltpu.SemaphoreType.DMA for _ in range(NBUF)],
      ],
  )
  def gather_kernel(idx_hbm, table_hbm, out_hbm, idx_v, bufs, gsems, wsems):
    wid = lax.axis_index("s") * NUM_CORES + lax.axis_index("c")
    base = wid * ROWS_PER_W
    s_base = wid * S_PER_W
    pltpu.sync_copy(idx_hbm.at[pl.ds(base, ROWS_PER_W)], idx_v)

    def gather_chunk(j, b):
      off = pl.multiple_of(j * CHUNK, CHUNK)
      return pltpu.make_async_copy(
          table_hbm.at[idx_v.at[pl.ds(off, CHUNK)]], bufs[b], gsems[b])

    def write_descs(j, b):
      s_off = s_base + j * (CHUNK // BATCH)
      return [
          pltpu.make_async_copy(
              bufs[b].at[pl.ds(k * BATCH, BATCH)], out_hbm.at[s_off + k],
              wsems[b])
          for k in range(CHUNK // BATCH)
      ]

    def write_chunk_start(j, b):
      for cp in write_descs(j, b):
        cp.start()

    def write_chunk_wait(j, b):
      for cp in write_descs(j, b):
        cp.wait()

    UNIFORM = N_CHUNKS - NBUF
    ROUNDS = UNIFORM // NBUF
    PEEL = UNIFORM % NBUF

    def step(j, b, b2, drain):
      gather_chunk(j, b).wait()
      write_chunk_start(j, b)
      if drain:
        write_chunk_wait(j + K - NBUF, b2)
      gather_chunk(j + K, b2).start()

    for j in range(K):
      gather_chunk(j, j % NBUF).start()
    for j in range(NBUF - K):
      step(j, j % NBUF, (j + K) % NBUF, drain=False)

    def body(p, carry):
      for i in range(NBUF):
        j = NBUF * p + (NBUF - K) + i
        step(j, (NBUF - K + i) % NBUF, i % NBUF, drain=True)
      return carry

    lax.fori_loop(0, ROUNDS, body, 0)

    for t in range(PEEL):
      j = NBUF * ROUNDS + (NBUF - K) + t
      step(j, j % NBUF, (j + K) % NBUF, drain=True)

    for j in range(N_CHUNKS - K, N_CHUNKS):
      gather_chunk(j, j % NBUF).wait()
      write_chunk_start(j, j % NBUF)
    for j in range(N_CHUNKS - NBUF, N_CHUNKS):
      write_chunk_wait(j, j % NBUF)

  return gather_kernel


_gather = _make_gather()


def kernel(input_ids, W):
  idx = input_ids.T.reshape(-1).astype(jnp.int32)
  return _gather(idx, W)


# speedup vs baseline: 1.1787x; 1.1787x over previous
import functools

import jax
import jax.numpy as jnp
from jax import lax
from jax.experimental import pallas as pl
from jax.experimental.pallas import tpu as pltpu
from jax.experimental.pallas import tpu_sc as plsc

VOCAB = 32000
HIDDEN = 2048
BATCH = 4
SEQ = 4096

NUM_CORES = 2
NUM_SUBCORES = 16
NUM_WORKERS = NUM_CORES * NUM_SUBCORES
ROWS = BATCH * SEQ
ROWS_PER_W = ROWS // NUM_WORKERS
CHUNK = 8
N_CHUNKS = ROWS_PER_W // CHUNK
NBUF = 4


def _make_gather():
  mesh = plsc.VectorSubcoreMesh(
      core_axis_name="c", subcore_axis_name="s",
      num_cores=NUM_CORES, num_subcores=NUM_SUBCORES)

  S_PER_W = ROWS_PER_W // BATCH

  @functools.partial(
      pl.kernel,
      out_type=jax.ShapeDtypeStruct((SEQ, BATCH, HIDDEN), jnp.float32),
      mesh=mesh,
      scratch_types=[
          pltpu.VMEM((ROWS_PER_W,), jnp.int32),
          [pltpu.VMEM((CHUNK, HIDDEN), jnp.float32) for _ in range(NBUF)],
          [pltpu.SemaphoreType.DMA for _ in range(NBUF)],
          [pltpu.SemaphoreType.DMA for _ in range(NBUF)],
      ],
  )
  def gather_kernel(idx_hbm, table_hbm, out_hbm, idx_v, bufs, gsems, wsems):
    wid = lax.axis_index("s") * NUM_CORES + lax.axis_index("c")
    base = wid * ROWS_PER_W
    s_base = wid * S_PER_W
    pltpu.sync_copy(idx_hbm.at[pl.ds(base, ROWS_PER_W)], idx_v)

    def gather_chunk(j, b):
      off = pl.multiple_of(j * CHUNK, CHUNK)
      return pltpu.make_async_copy(
          table_hbm.at[idx_v.at[pl.ds(off, CHUNK)]], bufs[b], gsems[b])

    def write_descs(j, b):
      s_off = s_base + j * (CHUNK // BATCH)
      return [
          pltpu.make_async_copy(
              bufs[b].at[pl.ds(k * BATCH, BATCH)], out_hbm.at[s_off + k],
              wsems[b])
          for k in range(CHUNK // BATCH)
      ]

    def write_chunk_start(j, b):
      for cp in write_descs(j, b):
        cp.start()

    def write_chunk_wait(j, b):
      for cp in write_descs(j, b):
        cp.wait()

    for b in range(NBUF):
      gather_chunk(b, b).start()

    def body(p, carry):
      for b in range(NBUF):
        j = p * NBUF + b
        gather_chunk(j, b).wait()
        write_chunk_start(j, b)
        write_chunk_wait(j, b)
        gather_chunk(j + NBUF, b).start()
      return carry

    lax.fori_loop(0, N_CHUNKS // NBUF - 1, body, 0)

    for b in range(NBUF):
      j = N_CHUNKS - NBUF + b
      gather_chunk(j, b).wait()
      write_chunk_start(j, b)
    for b in range(NBUF):
      write_chunk_wait(N_CHUNKS - NBUF + b, b)

  return gather_kernel


_gather = _make_gather()


def kernel(input_ids, W):
  idx = input_ids.T.reshape(-1).astype(jnp.int32)
  return _gather(idx, W)
